# Initial kernel scaffold; baseline (speedup 1.0000x reference)
import jax
import jax.numpy as jnp
from jax.experimental import pallas as pl


def kernel(x, edge_index, edge_weight, W0, W_ih, W_hh, b_ih, b_hh, lin_w, lin_b):
    raise NotImplementedError("write your pallas kernel here")


# speedup vs baseline: 11.1140x; 11.1140x over previous
import functools

import jax
import jax.numpy as jnp
from jax import lax
from jax.experimental import pallas as pl
from jax.experimental.pallas import tpu as pltpu
from jax.experimental.pallas import tpu_sc as plsc

N = 10000
E = 320000
D = 128

NC = 2
NS = 16
NW = NC * NS

NPAD = 10240
ROWS_PER_TILE = NPAD // NS

C = 128
EPAD = 327680
NCHUNK = EPAD // C
DEG_CH = NCHUNK // NS
EDGE_CH = NCHUNK // NW


def _rsqrt_newton(d):
    ib = lax.bitcast_convert_type(d, jnp.int32)
    y = lax.bitcast_convert_type(jnp.int32(0x5F3759DF) - (ib >> 1), jnp.float32)
    for _ in range(3):
        y = y * (1.5 - 0.5 * d * y * y)
    return y


def _sc_body(x_hbm, row_hbm, col_hbm, ew_hbm, t_out, dinv_out,
             t_sh, deg_sh, dinv_sh,
             bidx_v, bcol_v, bew_v,
             msg_v, zbuf_v, cbuf_v, gsem):
    cid = lax.axis_index("c")
    sid = lax.axis_index("s")
    wid = cid * NS + sid

    def _zmsg(i, _):
        for q in range(D // 16):
            msg_v[i, pl.ds(q * 16, 16)] = jnp.zeros((16,), jnp.float32)
        return _
    lax.fori_loop(0, C, _zmsg, 0)

    def _zrow(g, _):
        zbuf_v[pl.ds(g * 16, 16)] = jnp.zeros((16,), jnp.float32)
        return _
    lax.fori_loop(0, ROWS_PER_TILE // 16, _zrow, 0)

    pltpu.sync_copy(zbuf_v, deg_sh.at[pl.ds(sid * ROWS_PER_TILE, ROWS_PER_TILE)])
    for k in range(ROWS_PER_TILE // C):
        pltpu.sync_copy(msg_v, t_sh.at[pl.ds(sid * ROWS_PER_TILE + k * C, C)])

    plsc.subcore_barrier()

    def _dblk(b, _):
        dbase = pl.multiple_of(sid * DEG_CH + b * 8, 8)
        pltpu.sync_copy(col_hbm.at[pl.ds(dbase, 8)], bcol_v)
        pltpu.sync_copy(ew_hbm.at[pl.ds(dbase, 8)], bew_v)

        def _dchunk(j, _):
            pltpu.sync_copy(bew_v.at[j], deg_sh.at[bcol_v.at[j]], add=True)
            return _
        lax.fori_loop(0, 8, _dchunk, 0)
        return _
    lax.fori_loop(0, DEG_CH // 8, _dblk, 0)

    plsc.subcore_barrier()

    base = sid * ROWS_PER_TILE
    pltpu.sync_copy(deg_sh.at[pl.ds(base, ROWS_PER_TILE)], zbuf_v)

    def _dv(g, _):
        d = zbuf_v[pl.ds(g * 16, 16)] + 1.0
        zbuf_v[pl.ds(g * 16, 16)] = _rsqrt_newton(d)
        return _
    lax.fori_loop(0, ROWS_PER_TILE // 16, _dv, 0)
    pltpu.sync_copy(zbuf_v, dinv_sh.at[pl.ds(base, ROWS_PER_TILE)])

    @pl.when(cid == 0)
    def _():
        pltpu.sync_copy(zbuf_v, dinv_out.at[pl.ds(base, ROWS_PER_TILE)])

    plsc.subcore_barrier()

    def _eblk(b, _):
        ebase = pl.multiple_of(wid * EDGE_CH + b * 8, 8)
        pltpu.sync_copy(row_hbm.at[pl.ds(ebase, 8)], bidx_v)
        pltpu.sync_copy(col_hbm.at[pl.ds(ebase, 8)], bcol_v)
        pltpu.sync_copy(ew_hbm.at[pl.ds(ebase, 8)], bew_v)

        def _echunk(j, _):
            pltpu.async_copy(x_hbm.at[bidx_v.at[j]], msg_v, gsem).wait()
            pltpu.sync_copy(dinv_sh.at[bidx_v.at[j]], cbuf_v)
            for g in range(C // 16):
                wv = bew_v[j, pl.ds(g * 16, 16)]
                cf16 = wv * cbuf_v[pl.ds(g * 16, 16)]
                for l in range(16):
                    cf = cf16[l]
                    i = g * 16 + l
                    for q in range(D // 16):
                        msg_v[i, pl.ds(q * 16, 16)] = (
                            msg_v[i, pl.ds(q * 16, 16)] * cf)

            pltpu.sync_copy(msg_v, t_sh.at[bcol_v.at[j]], add=True)
            return _
        lax.fori_loop(0, 8, _echunk, 0)
        return _
    lax.fori_loop(0, EDGE_CH // 8, _eblk, 0)

    plsc.subcore_barrier()

    obase = cid * NPAD + sid * ROWS_PER_TILE
    pltpu.sync_copy(t_sh.at[pl.ds(sid * ROWS_PER_TILE, ROWS_PER_TILE)],
                    t_out.at[pl.ds(obase, ROWS_PER_TILE)])


@functools.partial(jax.jit, static_argnames=())
def _sc_sparse(x, row2d, col2d, ew2d):
    mesh = plsc.VectorSubcoreMesh(core_axis_name="c", subcore_axis_name="s")
    f = pl.kernel(
        _sc_body,
        out_type=[
            jax.ShapeDtypeStruct((NC * NPAD, D), jnp.float32),
            jax.ShapeDtypeStruct((NPAD,), jnp.float32),
        ],
        mesh=mesh,
        compiler_params=pltpu.CompilerParams(needs_layout_passes=False),
        scratch_types=[
            pltpu.VMEM_SHARED((NPAD, D), jnp.float32),
            pltpu.VMEM_SHARED((NPAD,), jnp.float32),
            pltpu.VMEM_SHARED((NPAD,), jnp.float32),
            pltpu.VMEM((8, C), jnp.int32),
            pltpu.VMEM((8, C), jnp.int32),
            pltpu.VMEM((8, C), jnp.float32),
            pltpu.VMEM((C, D), jnp.float32),
            pltpu.VMEM((ROWS_PER_TILE,), jnp.float32),
            pltpu.VMEM((C,), jnp.float32),
            pltpu.SemaphoreType.DMA,
        ],
    )
    return f(x, row2d, col2d, ew2d)


BR = 1000


def _tc_body(x_ref, t0_ref, t1_ref, dinv_ref, W0_ref, Wih_ref, Whh_ref,
             bih_ref, bhh_ref, lw_ref, lb_ref, out_ref, W_s):
    @pl.when(pl.program_id(0) == 0)
    def _():
        W0 = W0_ref[...]
        gi = jnp.dot(W0, Wih_ref[...].T,
                     preferred_element_type=jnp.float32) + bih_ref[...]
        gh = jnp.dot(W0, Whh_ref[...].T,
                     preferred_element_type=jnp.float32) + bhh_ref[...]
        i_r, i_z, i_n = jnp.split(gi, 3, axis=1)
        h_r, h_z, h_n = jnp.split(gh, 3, axis=1)
        r = jax.nn.sigmoid(i_r + h_r)
        z = jax.nn.sigmoid(i_z + h_z)
        n = jnp.tanh(i_n + r * h_n)
        W_s[...] = (1.0 - z) * n + z * W0

    dinv = dinv_ref[...]
    u = dinv * (t0_ref[0] + t1_ref[0] + dinv * x_ref[...])
    agg = jnp.dot(u, W_s[...], preferred_element_type=jnp.float32)
    h = jnp.maximum(agg, 0.0)
    out_ref[...] = jnp.sum(h * lw_ref[...], axis=1, keepdims=True) + lb_ref[0]


@jax.jit
def _tc_dense(x, t3d, dinv2d, W0, W_ih, W_hh, b_ih, b_hh, lin_w, lin_b):
    grid = N // BR
    return pl.pallas_call(
        _tc_body,
        grid=(grid,),
        in_specs=[
            pl.BlockSpec((BR, D), lambda i: (i, 0)),
            pl.BlockSpec((1, BR, D), lambda i: (0, i, 0)),
            pl.BlockSpec((1, BR, D), lambda i: (1, i, 0)),
            pl.BlockSpec((BR, 1), lambda i: (i, 0)),
            pl.BlockSpec((D, D), lambda i: (0, 0)),
            pl.BlockSpec((3 * D, D), lambda i: (0, 0)),
            pl.BlockSpec((3 * D, D), lambda i: (0, 0)),
            pl.BlockSpec((3 * D,), lambda i: (0,)),
            pl.BlockSpec((3 * D,), lambda i: (0,)),
            pl.BlockSpec((1, D), lambda i: (0, 0)),
            pl.BlockSpec(memory_space=pltpu.SMEM),
        ],
        out_specs=pl.BlockSpec((BR, 1), lambda i: (i, 0)),
        out_shape=jax.ShapeDtypeStruct((N, 1), jnp.float32),
        scratch_shapes=[pltpu.VMEM((D, D), jnp.float32)],
    )(x, t3d, t3d, dinv2d, W0, W_ih, W_hh, b_ih, b_hh, lin_w, lin_b)


def kernel(x, edge_index, edge_weight, W0, W_ih, W_hh, b_ih, b_hh, lin_w, lin_b):
    pad = EPAD - E
    zi = jnp.zeros((pad,), jnp.int32)
    row2d = jnp.concatenate([edge_index[0], zi]).reshape(NCHUNK, C)
    col2d = jnp.concatenate([edge_index[1], zi]).reshape(NCHUNK, C)
    ew2d = jnp.concatenate(
        [edge_weight, jnp.zeros((pad,), jnp.float32)]).reshape(NCHUNK, C)
    t_flat, dinv = _sc_sparse(x, row2d, col2d, ew2d)
    t3d = t_flat.reshape(NC, NPAD, D)
    dinv2d = dinv[:N, None]
    return _tc_dense(x, t3d, dinv2d, W0, W_ih, W_hh, b_ih, b_hh,
                     lin_w, lin_b)
